# baseline (speedup 1.0000x reference)
import functools

import jax
import jax.numpy as jnp
from jax.experimental import pallas as pl
from jax.experimental.pallas import tpu as pltpu


def _pick_block(n, target):
    b = min(n, target)
    while n % b != 0:
        b -= 1
    return b


def _relu(x):
    return jnp.maximum(x, 0.0)


def _mlp1_kernel(x_ref, w1_ref, b1_ref, w2_ref, b2_ref, o_ref):
    h = _relu(jnp.dot(x_ref[...], w1_ref[...],
                      preferred_element_type=jnp.float32) + b1_ref[...])
    o_ref[...] = _relu(jnp.dot(h, w2_ref[...],
                               preferred_element_type=jnp.float32) + b2_ref[...])


def _voxmm_kernel(x_ref, w_ref, b_ref, o_ref):
    o_ref[...] = _relu(jnp.dot(x_ref[...], w_ref[...],
                               preferred_element_type=jnp.float32) + b_ref[...])


def _mlp2_kernel(pg_ref, pf_ref, w3a_ref, w3b_ref, b3_ref, w4_ref, b4_ref, o_ref):
    h = jnp.dot(pg_ref[...], w3a_ref[...], preferred_element_type=jnp.float32)
    h = h + jnp.dot(pf_ref[...], w3b_ref[...], preferred_element_type=jnp.float32)
    h = _relu(h + b3_ref[...])
    o_ref[...] = _relu(jnp.dot(h, w4_ref[...],
                               preferred_element_type=jnp.float32) + b4_ref[...])


def _segmax_kernel(idx_ref, x_ref, o_ref, *, rows, group):
    @pl.when(pl.program_id(0) == 0)
    def _init():
        o_ref[...] = jnp.zeros_like(o_ref)

    def body(g, carry):
        r0 = g * group
        v0 = idx_ref[0, 0, r0]
        v1 = idx_ref[0, 0, r0 + group - 1]

        @pl.when(v0 == v1)
        def _fast():
            m = jnp.max(x_ref[pl.ds(r0, group), :], axis=0, keepdims=True)
            o_ref[pl.ds(v0, 1), :] = jnp.maximum(o_ref[pl.ds(v0, 1), :], m)

        @pl.when(v0 != v1)
        def _slow():
            def rb(r, c):
                v = idx_ref[0, 0, r]
                o_ref[pl.ds(v, 1), :] = jnp.maximum(
                    o_ref[pl.ds(v, 1), :], x_ref[pl.ds(r, 1), :])
                return c
            jax.lax.fori_loop(r0, r0 + group, rb, 0)

        return carry

    jax.lax.fori_loop(0, rows // group, body, 0)


def _gather_kernel(idx_ref, vox_ref, o_ref, *, rows, group):
    def body(g, carry):
        r0 = g * group
        v0 = idx_ref[0, 0, r0]
        v1 = idx_ref[0, 0, r0 + group - 1]

        @pl.when(v0 == v1)
        def _fast():
            row = vox_ref[pl.ds(v0, 1), :]
            o_ref[pl.ds(r0, group), :] = jnp.broadcast_to(
                row, (group, row.shape[1]))

        @pl.when(v0 != v1)
        def _slow():
            def rb(r, c):
                v = idx_ref[0, 0, r]
                o_ref[pl.ds(r, 1), :] = vox_ref[pl.ds(v, 1), :]
                return c
            jax.lax.fori_loop(r0, r0 + group, rb, 0)

        return carry

    jax.lax.fori_loop(0, rows // group, body, 0)


def _mlp1(inp, w1t, b1, w2t, b2):
    n, cin = inp.shape
    gf = w1t.shape[1]
    f1 = w2t.shape[1]
    bn = _pick_block(n, 640)
    grid = n // bn
    return pl.pallas_call(
        _mlp1_kernel,
        grid=(grid,),
        in_specs=[
            pl.BlockSpec((bn, cin), lambda i: (i, 0)),
            pl.BlockSpec((cin, gf), lambda i: (0, 0)),
            pl.BlockSpec((1, gf), lambda i: (0, 0)),
            pl.BlockSpec((gf, f1), lambda i: (0, 0)),
            pl.BlockSpec((1, f1), lambda i: (0, 0)),
        ],
        out_specs=pl.BlockSpec((bn, f1), lambda i: (i, 0)),
        out_shape=jax.ShapeDtypeStruct((n, f1), jnp.float32),
    )(inp, w1t, b1.reshape(1, -1), w2t, b2.reshape(1, -1))


def _voxmm(x, wt, b):
    v, fin = x.shape
    fout = wt.shape[1]
    bv = _pick_block(v, 1000)
    grid = v // bv
    return pl.pallas_call(
        _voxmm_kernel,
        grid=(grid,),
        in_specs=[
            pl.BlockSpec((bv, fin), lambda i: (i, 0)),
            pl.BlockSpec((fin, fout), lambda i: (0, 0)),
            pl.BlockSpec((1, fout), lambda i: (0, 0)),
        ],
        out_specs=pl.BlockSpec((bv, fout), lambda i: (i, 0)),
        out_shape=jax.ShapeDtypeStruct((v, fout), jnp.float32),
    )(x, wt, b.reshape(1, -1))


def _segmax(x, idx3, v, group):
    n, f = x.shape
    nb, _, br = idx3.shape
    return pl.pallas_call(
        functools.partial(_segmax_kernel, rows=br, group=group),
        grid=(nb,),
        in_specs=[
            pl.BlockSpec((1, 1, br), lambda i: (i, 0, 0),
                         memory_space=pltpu.SMEM),
            pl.BlockSpec((br, f), lambda i: (i, 0)),
        ],
        out_specs=pl.BlockSpec((v, f), lambda i: (0, 0)),
        out_shape=jax.ShapeDtypeStruct((v, f), jnp.float32),
    )(idx3, x)


def _gather(vox, idx3, n, group):
    v, f = vox.shape
    nb, _, br = idx3.shape
    return pl.pallas_call(
        functools.partial(_gather_kernel, rows=br, group=group),
        grid=(nb,),
        in_specs=[
            pl.BlockSpec((1, 1, br), lambda i: (i, 0, 0),
                         memory_space=pltpu.SMEM),
            pl.BlockSpec((v, f), lambda i: (0, 0)),
        ],
        out_specs=pl.BlockSpec((br, f), lambda i: (i, 0)),
        out_shape=jax.ShapeDtypeStruct((n, f), jnp.float32),
    )(idx3, vox)


def _mlp2(pg, pf, w3at, w3bt, b3, w4t, b4):
    n, f1 = pf.shape
    oc = w4t.shape[1]
    bn = _pick_block(n, 640)
    grid = n // bn
    return pl.pallas_call(
        _mlp2_kernel,
        grid=(grid,),
        in_specs=[
            pl.BlockSpec((bn, f1), lambda i: (i, 0)),
            pl.BlockSpec((bn, f1), lambda i: (i, 0)),
            pl.BlockSpec((f1, oc), lambda i: (0, 0)),
            pl.BlockSpec((f1, oc), lambda i: (0, 0)),
            pl.BlockSpec((1, oc), lambda i: (0, 0)),
            pl.BlockSpec((oc, oc), lambda i: (0, 0)),
            pl.BlockSpec((1, oc), lambda i: (0, 0)),
        ],
        out_specs=pl.BlockSpec((bn, oc), lambda i: (i, 0)),
        out_shape=jax.ShapeDtypeStruct((n, oc), jnp.float32),
    )(pg, pf, w3at, w3bt, b3.reshape(1, -1), w4t, b4.reshape(1, -1))


def _forward(inp_feat, vox2point_idx, W1, b1, W2, b2, Wv1, bv1,
             W3, b3, W4, b4, Wv2, bv2, v):
    n = inp_feat.shape[0]
    f1 = W2.shape[0]
    idx = vox2point_idx.astype(jnp.int32)

    br = _pick_block(n, 3200)
    idx3 = idx.reshape(n // br, 1, br)

    cin = inp_feat.shape[1]
    cpad = (-cin) % 8
    inp_p = jnp.pad(inp_feat, ((0, 0), (0, cpad)))
    w1t = jnp.pad(W1.T, ((0, cpad), (0, 0)))

    pf2 = _mlp1(inp_p, w1t, b1, W2.T, b2)
    vox1 = _segmax(pf2, idx3, v, group=8)
    vox1r = _voxmm(vox1, Wv1.T, bv1)
    pgf = _gather(vox1r, idx3, n, group=8)
    pf5 = _mlp2(pgf, pf2, W3[:, :f1].T, W3[:, f1:].T, b3, W4.T, b4)
    vox2 = _segmax(pf5, idx3, v, group=8)
    return _voxmm(vox2, Wv2.T, bv2)


def kernel(inp_feat, vox2point_idx, W1, b1, W2, b2, Wv1, bv1,
           W3, b3, W4, b4, Wv2, bv2):
    return _forward(inp_feat, vox2point_idx, W1, b1, W2, b2, Wv1, bv1,
                    W3, b3, W4, b4, Wv2, bv2, v=10000)


# speedup vs baseline: 1.5731x; 1.5731x over previous
import functools

import jax
import jax.numpy as jnp
from jax import lax
from jax.experimental import pallas as pl
from jax.experimental.pallas import tpu as pltpu
from jax.experimental.pallas import tpu_sc as plsc

_SC_NC = 2
_SC_NS = 16
_SC_NW = _SC_NC * _SC_NS


def _pick_block(n, target):
    b = min(n, target)
    while n % b != 0:
        b -= 1
    return b


def _relu(x):
    return jnp.maximum(x, 0.0)


def _mlp1_kernel(x_ref, w1_ref, b1_ref, w2_ref, b2_ref, o_ref):
    h = _relu(jnp.dot(x_ref[...], w1_ref[...],
                      preferred_element_type=jnp.float32) + b1_ref[...])
    o_ref[...] = _relu(jnp.dot(h, w2_ref[...],
                               preferred_element_type=jnp.float32) + b2_ref[...])


def _voxmm_kernel(x_ref, w_ref, b_ref, o_ref):
    o_ref[...] = _relu(jnp.dot(x_ref[...], w_ref[...],
                               preferred_element_type=jnp.float32) + b_ref[...])


def _mlp2_kernel(pg_ref, pf_ref, w3a_ref, w3b_ref, b3_ref, w4_ref, b4_ref, o_ref):
    h = jnp.dot(pg_ref[...], w3a_ref[...], preferred_element_type=jnp.float32)
    h = h + jnp.dot(pf_ref[...], w3b_ref[...], preferred_element_type=jnp.float32)
    h = _relu(h + b3_ref[...])
    o_ref[...] = _relu(jnp.dot(h, w4_ref[...],
                               preferred_element_type=jnp.float32) + b4_ref[...])


def _segmax_kernel(idx_ref, x_ref, o_ref, *, rows, group):
    @pl.when(pl.program_id(0) == 0)
    def _init():
        o_ref[...] = jnp.zeros_like(o_ref)

    def body(g, carry):
        r0 = g * group
        v0 = idx_ref[0, 0, r0]
        v1 = idx_ref[0, 0, r0 + group - 1]

        @pl.when(v0 == v1)
        def _fast():
            m = jnp.max(x_ref[pl.ds(r0, group), :], axis=0, keepdims=True)
            o_ref[pl.ds(v0, 1), :] = jnp.maximum(o_ref[pl.ds(v0, 1), :], m)

        @pl.when(v0 != v1)
        def _slow():
            def rb(r, c):
                v = idx_ref[0, 0, r]
                o_ref[pl.ds(v, 1), :] = jnp.maximum(
                    o_ref[pl.ds(v, 1), :], x_ref[pl.ds(r, 1), :])
                return c
            jax.lax.fori_loop(r0, r0 + group, rb, 0)

        return carry

    jax.lax.fori_loop(0, rows // group, body, 0)


def _gather_kernel(idx_ref, vox_ref, o_ref, *, rows, group):
    def body(g, carry):
        r0 = g * group
        v0 = idx_ref[0, 0, r0]
        v1 = idx_ref[0, 0, r0 + group - 1]

        @pl.when(v0 == v1)
        def _fast():
            row = vox_ref[pl.ds(v0, 1), :]
            o_ref[pl.ds(r0, group), :] = jnp.broadcast_to(
                row, (group, row.shape[1]))

        @pl.when(v0 != v1)
        def _slow():
            def rb(r, c):
                v = idx_ref[0, 0, r]
                o_ref[pl.ds(r, 1), :] = vox_ref[pl.ds(v, 1), :]
                return c
            jax.lax.fori_loop(r0, r0 + group, rb, 0)

        return carry

    jax.lax.fori_loop(0, rows // group, body, 0)


def _mlp1(inp, w1t, b1, w2t, b2):
    n, cin = inp.shape
    gf = w1t.shape[1]
    f1 = w2t.shape[1]
    bn = _pick_block(n, 640)
    grid = n // bn
    return pl.pallas_call(
        _mlp1_kernel,
        grid=(grid,),
        in_specs=[
            pl.BlockSpec((bn, cin), lambda i: (i, 0)),
            pl.BlockSpec((cin, gf), lambda i: (0, 0)),
            pl.BlockSpec((1, gf), lambda i: (0, 0)),
            pl.BlockSpec((gf, f1), lambda i: (0, 0)),
            pl.BlockSpec((1, f1), lambda i: (0, 0)),
        ],
        out_specs=pl.BlockSpec((bn, f1), lambda i: (i, 0)),
        out_shape=jax.ShapeDtypeStruct((n, f1), jnp.float32),
    )(inp, w1t, b1.reshape(1, -1), w2t, b2.reshape(1, -1))


def _voxmm(x, wt, b):
    v, fin = x.shape
    fout = wt.shape[1]
    bv = _pick_block(v, 1000)
    grid = v // bv
    return pl.pallas_call(
        _voxmm_kernel,
        grid=(grid,),
        in_specs=[
            pl.BlockSpec((bv, fin), lambda i: (i, 0)),
            pl.BlockSpec((fin, fout), lambda i: (0, 0)),
            pl.BlockSpec((1, fout), lambda i: (0, 0)),
        ],
        out_specs=pl.BlockSpec((bv, fout), lambda i: (i, 0)),
        out_shape=jax.ShapeDtypeStruct((v, fout), jnp.float32),
    )(x, wt, b.reshape(1, -1))


def _segmax(x, idx3, v, group):
    n, f = x.shape
    nb, _, br = idx3.shape
    return pl.pallas_call(
        functools.partial(_segmax_kernel, rows=br, group=group),
        grid=(nb,),
        in_specs=[
            pl.BlockSpec((1, 1, br), lambda i: (i, 0, 0),
                         memory_space=pltpu.SMEM),
            pl.BlockSpec((br, f), lambda i: (i, 0)),
        ],
        out_specs=pl.BlockSpec((v, f), lambda i: (0, 0)),
        out_shape=jax.ShapeDtypeStruct((v, f), jnp.float32),
    )(idx3, x)


def _gather(vox, idx3, n, group):
    v, f = vox.shape
    nb, _, br = idx3.shape
    return pl.pallas_call(
        functools.partial(_gather_kernel, rows=br, group=group),
        grid=(nb,),
        in_specs=[
            pl.BlockSpec((1, 1, br), lambda i: (i, 0, 0),
                         memory_space=pltpu.SMEM),
            pl.BlockSpec((v, f), lambda i: (0, 0)),
        ],
        out_specs=pl.BlockSpec((br, f), lambda i: (i, 0)),
        out_shape=jax.ShapeDtypeStruct((n, f), jnp.float32),
    )(idx3, vox)


def _sc_gather(vox, idx, n):
    v, f = vox.shape
    cpw = n // _SC_NW
    ch = 400
    while cpw % ch != 0:
        ch -= 8
    nch = cpw // ch
    mesh = plsc.VectorSubcoreMesh(core_axis_name="c", subcore_axis_name="s")

    def body(vox_hbm, idx_hbm, out_hbm, idx_v, rows_v, sem):
        wid = lax.axis_index("s") * _SC_NC + lax.axis_index("c")
        base = wid * cpw
        pltpu.sync_copy(idx_hbm.at[pl.ds(base, cpw)], idx_v)

        def start(c, b):
            return pltpu.async_copy(
                vox_hbm.at[idx_v.at[pl.ds(c * ch, ch)]], rows_v.at[b], sem)

        cps = [start(0, 0), None]
        for c in range(nch):
            b = c % 2
            if c + 1 < nch:
                cps[1 - b] = start(c + 1, 1 - b)
            cps[b].wait()
            pltpu.sync_copy(rows_v.at[b],
                            out_hbm.at[pl.ds(base + c * ch, ch)])

    return pl.kernel(
        body,
        out_type=jax.ShapeDtypeStruct((n, f), jnp.float32),
        mesh=mesh,
        scratch_types=[
            pltpu.VMEM((cpw,), jnp.int32),
            pltpu.VMEM((2, ch, f), jnp.float32),
            pltpu.SemaphoreType.DMA,
        ],
    )(vox, idx)


def _sc_segmax(x, idx, zeros_flat, v):
    n, f = x.shape
    c = n // _SC_NW
    fj = f // 16
    if f == 128:
        r, sw = 400, 128
    else:
        r, sw = 80, 256
    nwin = c // r
    pieces = []
    p = sw
    while p >= 1:
        pieces.append(p)
        p //= 2
    zpieces = []
    p = 8192
    while p >= 1:
        zpieces.append(p)
        p //= 2
    mesh = plsc.VectorSubcoreMesh(core_axis_name="c", subcore_axis_name="s")

    def body(x_hbm, idx_hbm, z_hbm, out_hbm, hacc_hbm, prev_hbm,
             xbf, ibf, win, bnd, sx0, sx1, si0, si1):
        wid = lax.axis_index("s") * _SC_NC + lax.axis_index("c")
        base = wid * c
        zero16 = jnp.zeros((16,), jnp.float32)

        def zero_rows(lo, hi):
            def zrow(i, carry):
                for j in range(fj):
                    win[pl.ds(i * f + 16 * j, 16)] = zero16
                return carry
            lax.fori_loop(lo, hi, zrow, 0)

        zero_rows(0, sw + 1)

        pltpu.sync_copy(
            idx_hbm.at[pl.ds(pl.multiple_of(jnp.maximum(base - 16, 0), 16),
                             16)], bnd)
        prev_last = jnp.where(wid == 0, jnp.int32(-1), bnd[...][15])
        own_lo = prev_last + 1
        pltpu.sync_copy(
            idx_hbm.at[pl.ds(pl.multiple_of(base + c - 16, 16), 16)], bnd)
        last_mine = bnd[...][15]
        own_hi = jnp.where(wid == _SC_NW - 1, jnp.int32(v), last_mine + 1)

        def zfill(lo, hi):
            ln = jnp.clip(jnp.minimum(hi, own_hi) - lo, 0, v)
            off = jnp.int32(0)
            for pc in zpieces:
                pred = (ln & pc) != 0

                @pl.when(pred)
                def _(off=off, pc=pc):
                    pltpu.sync_copy(
                        z_hbm.at[pl.ds(0, pc * f)],
                        out_hbm.at[pl.ds(
                            pl.multiple_of((lo + off) * f, 128), pc * f)])
                off = off + jnp.where(pred, pc, 0)

        def drain_zero(w0):
            ln = jnp.clip(own_hi - w0, 0, sw)
            off = jnp.int32(0)
            for pc in pieces:
                pred = (ln & pc) != 0

                @pl.when(pred)
                def _(off=off, pc=pc):
                    pltpu.sync_copy(
                        win.at[pl.ds(pl.multiple_of(off * f, 128), pc * f)],
                        out_hbm.at[pl.ds(
                            pl.multiple_of((w0 + off) * f, 128), pc * f)])
                off = off + jnp.where(pred, pc, 0)
            zero_rows(0, sw)

        def rmw(rowscalar, accs):
            o = rowscalar * f
            for j in range(fj):
                cur = win[pl.ds(o + 16 * j, 16)]
                win[pl.ds(o + 16 * j, 16)] = jnp.maximum(cur, accs[j])

        def xrow(boff, row):
            return [xbf[pl.ds(boff + row * f + 16 * j, 16)]
                    for j in range(fj)]

        def group_body(b):
            boff = b * (r * f)

            def gb(g, w0):
                iv = ibf[pl.ds(b * r + g * 16, 16)]
                g0 = iv[0]
                g15 = iv[15]
                row0 = g * 16

                def cls_u(w0):
                    accs = xrow(boff, row0)
                    for rr in range(1, 16):
                        nxt = xrow(boff, row0 + rr)
                        accs = [jnp.maximum(a, bq)
                                for a, bq in zip(accs, nxt)]
                    trow = jnp.where(g0 == prev_last, sw, g0 - w0)
                    rmw(trow, accs)
                    return w0

                def general(w0):
                    def rnd(q, carry):
                        w, active = carry

                        @pl.when(active)
                        def _(w=w):
                            for rr in range(16):
                                s = iv[rr]
                                isp = s == prev_last
                                ok = isp | ((s >= own_lo) & (s >= w)
                                            & (s < w + sw))

                                @pl.when(ok)
                                def _(s=s, rr=rr, w=w, isp=isp):
                                    rmw(jnp.where(isp, sw, s - w),
                                        xrow(boff, row0 + rr))
                        cross = active & (g15 >= w + sw)

                        @pl.when(cross)
                        def _(w=w):
                            drain_zero(w)
                        nw0 = g15
                        for rr in range(14, -1, -1):
                            nw0 = jnp.where(iv[rr] >= w + sw, iv[rr], nw0)

                        @pl.when(cross)
                        def _(w=w, nw0=nw0):
                            zfill(w + sw, nw0)
                        return (jnp.where(cross, nw0, w), cross)

                    w, _ = lax.fori_loop(0, 16, rnd,
                                         (w0, jnp.bool_(True)))
                    return w

                is_u = (g0 == g15) & (
                    (g0 == prev_last)
                    | ((g0 >= own_lo) & (g15 < w0 + sw)))
                return lax.cond(is_u, cls_u, general, w0)
            return gb

        def start_w(wi, b, sx, si):
            pltpu.async_copy(
                x_hbm.at[pl.ds(pl.multiple_of((base + wi * r) * f, 128),
                               r * f)],
                xbf.at[pl.ds(b * (r * f), r * f)], sx)
            pltpu.async_copy(
                idx_hbm.at[pl.ds(pl.multiple_of(base + wi * r, 16), r)],
                ibf.at[pl.ds(b * r, r)], si)

        def wait_w(wi, b, sx, si):
            pltpu.make_async_copy(
                x_hbm.at[pl.ds(pl.multiple_of((base + wi * r) * f, 128),
                               r * f)],
                xbf.at[pl.ds(b * (r * f), r * f)], sx).wait()
            pltpu.make_async_copy(
                idx_hbm.at[pl.ds(pl.multiple_of(base + wi * r, 16), r)],
                ibf.at[pl.ds(b * r, r)], si).wait()

        start_w(0, 0, sx0, si0)
        start_w(1, 1, sx1, si1)

        def wbody(wi, w0):
            def even(w0):
                wait_w(wi, 0, sx0, si0)
                w0 = lax.fori_loop(0, r // 16, group_body(0), w0)

                @pl.when(wi + 2 < nwin)
                def _():
                    start_w(wi + 2, 0, sx0, si0)
                return w0

            def odd(w0):
                wait_w(wi, 1, sx1, si1)
                w0 = lax.fori_loop(0, r // 16, group_body(1), w0)

                @pl.when(wi + 2 < nwin)
                def _():
                    start_w(wi + 2, 1, sx1, si1)
                return w0

            return lax.cond(wi % 2 == 0, even, odd, w0)

        w0 = lax.fori_loop(0, nwin, wbody, own_lo)

        drain_zero(w0)
        zfill(w0 + sw, own_hi)
        pltpu.sync_copy(
            win.at[pl.ds(pl.multiple_of(sw * f, 128), f)],
            hacc_hbm.at[pl.ds(pl.multiple_of(wid * f, 128), f)])
        bnd[pl.ds(0, 16)] = jnp.full((16,), prev_last, jnp.int32)
        pltpu.sync_copy(bnd,
                        prev_hbm.at[pl.ds(pl.multiple_of(wid * 16, 16), 16)])

    outs = pl.kernel(
        body,
        out_type=(
            jax.ShapeDtypeStruct((v * f,), jnp.float32),
            jax.ShapeDtypeStruct((_SC_NW * f,), jnp.float32),
            jax.ShapeDtypeStruct((_SC_NW * 16,), jnp.int32),
        ),
        mesh=mesh,
        scratch_types=[
            pltpu.VMEM((2 * r * f,), jnp.float32),
            pltpu.VMEM((2 * r,), jnp.int32),
            pltpu.VMEM(((sw + 1) * f,), jnp.float32),
            pltpu.VMEM((16,), jnp.int32),
            pltpu.SemaphoreType.DMA,
            pltpu.SemaphoreType.DMA,
            pltpu.SemaphoreType.DMA,
            pltpu.SemaphoreType.DMA,
        ],
    )(x.reshape(-1), idx, zeros_flat)
    return outs[0].reshape(v, f), outs[1].reshape(_SC_NW, f), outs[2]


def _fold_kernel(prev_ref, hacc_ref, vox_ref, o_ref, *, nw):
    o_ref[...] = vox_ref[...]

    def fb(k, carry):
        pv = prev_ref[0, 0, k * 16]

        @pl.when(pv >= 0)
        def _():
            o_ref[pl.ds(pv, 1), :] = jnp.maximum(
                o_ref[pl.ds(pv, 1), :], hacc_ref[pl.ds(k, 1), :])
        return carry
    lax.fori_loop(0, nw, fb, 0)


def _fold(prevs, haccs, vox):
    v, f = vox.shape
    nw = haccs.shape[0]
    return pl.pallas_call(
        functools.partial(_fold_kernel, nw=nw),
        grid=(1,),
        in_specs=[
            pl.BlockSpec((1, 1, nw * 16), lambda i: (0, 0, 0),
                         memory_space=pltpu.SMEM),
            pl.BlockSpec((nw, f), lambda i: (0, 0)),
            pl.BlockSpec((v, f), lambda i: (0, 0)),
        ],
        out_specs=pl.BlockSpec((v, f), lambda i: (0, 0)),
        out_shape=jax.ShapeDtypeStruct((v, f), jnp.float32),
    )(prevs.reshape(1, 1, nw * 16), haccs, vox)


def _mlp2(pg, pf, w3at, w3bt, b3, w4t, b4):
    n, f1 = pf.shape
    oc = w4t.shape[1]
    bn = _pick_block(n, 640)
    grid = n // bn
    return pl.pallas_call(
        _mlp2_kernel,
        grid=(grid,),
        in_specs=[
            pl.BlockSpec((bn, f1), lambda i: (i, 0)),
            pl.BlockSpec((bn, f1), lambda i: (i, 0)),
            pl.BlockSpec((f1, oc), lambda i: (0, 0)),
            pl.BlockSpec((f1, oc), lambda i: (0, 0)),
            pl.BlockSpec((1, oc), lambda i: (0, 0)),
            pl.BlockSpec((oc, oc), lambda i: (0, 0)),
            pl.BlockSpec((1, oc), lambda i: (0, 0)),
        ],
        out_specs=pl.BlockSpec((bn, oc), lambda i: (i, 0)),
        out_shape=jax.ShapeDtypeStruct((n, oc), jnp.float32),
    )(pg, pf, w3at, w3bt, b3.reshape(1, -1), w4t, b4.reshape(1, -1))


def _forward(inp_feat, vox2point_idx, W1, b1, W2, b2, Wv1, bv1,
             W3, b3, W4, b4, Wv2, bv2, v):
    n = inp_feat.shape[0]
    f1 = W2.shape[0]
    idx = vox2point_idx.astype(jnp.int32)

    br = _pick_block(n, 3200)
    idx3 = idx.reshape(n // br, 1, br)

    cin = inp_feat.shape[1]
    cpad = (-cin) % 8
    inp_p = jnp.pad(inp_feat, ((0, 0), (0, cpad)))
    w1t = jnp.pad(W1.T, ((0, cpad), (0, 0)))

    pf2 = _mlp1(inp_p, w1t, b1, W2.T, b2)
    z128 = jnp.zeros((v * f1,), jnp.float32)
    vm1, ha1, pv1 = _sc_segmax(pf2, idx, z128, v)
    vox1 = _fold(pv1, ha1, vm1)
    vox1r = _voxmm(vox1, Wv1.T, bv1)
    pgf = _sc_gather(vox1r, idx, n)
    pf5 = _mlp2(pgf, pf2, W3[:, :f1].T, W3[:, f1:].T, b3, W4.T, b4)
    oc = W3.shape[0]
    z256 = jnp.zeros((v * oc,), jnp.float32)
    vm2, ha2, pv2 = _sc_segmax(pf5, idx, z256, v)
    vox2 = _fold(pv2, ha2, vm2)
    return _voxmm(vox2, Wv2.T, bv2)


def kernel(inp_feat, vox2point_idx, W1, b1, W2, b2, Wv1, bv1,
           W3, b3, W4, b4, Wv2, bv2):
    return _forward(inp_feat, vox2point_idx, W1, b1, W2, b2, Wv1, bv1,
                    W3, b3, W4, b4, Wv2, bv2, v=10000)


# speedup vs baseline: 1.7207x; 1.0938x over previous
import functools

import jax
import jax.numpy as jnp
from jax import lax
from jax.experimental import pallas as pl
from jax.experimental.pallas import tpu as pltpu
from jax.experimental.pallas import tpu_sc as plsc

_SC_NC = 2
_SC_NS = 16
_SC_NW = _SC_NC * _SC_NS


def _pick_block(n, target):
    b = min(n, target)
    while n % b != 0:
        b -= 1
    return b


def _relu(x):
    return jnp.maximum(x, 0.0)


def _mlp1_kernel(x_ref, w1_ref, b1_ref, w2_ref, b2_ref, o_ref):
    h = _relu(jnp.dot(x_ref[...], w1_ref[...],
                      preferred_element_type=jnp.float32) + b1_ref[...])
    o_ref[...] = _relu(jnp.dot(h, w2_ref[...],
                               preferred_element_type=jnp.float32) + b2_ref[...])


def _voxmm_kernel(x_ref, w_ref, b_ref, o_ref):
    o_ref[...] = _relu(jnp.dot(x_ref[...], w_ref[...],
                               preferred_element_type=jnp.float32) + b_ref[...])


def _mlp2_kernel(pg_ref, pf_ref, w3a_ref, w3b_ref, b3_ref, w4_ref, b4_ref, o_ref):
    h = jnp.dot(pg_ref[...], w3a_ref[...], preferred_element_type=jnp.float32)
    h = h + jnp.dot(pf_ref[...], w3b_ref[...], preferred_element_type=jnp.float32)
    h = _relu(h + b3_ref[...])
    o_ref[...] = _relu(jnp.dot(h, w4_ref[...],
                               preferred_element_type=jnp.float32) + b4_ref[...])


def _segmax_kernel(idx_ref, x_ref, o_ref, *, rows, group):
    @pl.when(pl.program_id(0) == 0)
    def _init():
        o_ref[...] = jnp.zeros_like(o_ref)

    def body(g, carry):
        r0 = g * group
        v0 = idx_ref[0, 0, r0]
        v1 = idx_ref[0, 0, r0 + group - 1]

        @pl.when(v0 == v1)
        def _fast():
            m = jnp.max(x_ref[pl.ds(r0, group), :], axis=0, keepdims=True)
            o_ref[pl.ds(v0, 1), :] = jnp.maximum(o_ref[pl.ds(v0, 1), :], m)

        @pl.when(v0 != v1)
        def _slow():
            def rb(r, c):
                v = idx_ref[0, 0, r]
                o_ref[pl.ds(v, 1), :] = jnp.maximum(
                    o_ref[pl.ds(v, 1), :], x_ref[pl.ds(r, 1), :])
                return c
            jax.lax.fori_loop(r0, r0 + group, rb, 0)

        return carry

    jax.lax.fori_loop(0, rows // group, body, 0)


def _gather_kernel(idx_ref, vox_ref, o_ref, *, rows, group):
    def body(g, carry):
        r0 = g * group
        v0 = idx_ref[0, 0, r0]
        v1 = idx_ref[0, 0, r0 + group - 1]

        @pl.when(v0 == v1)
        def _fast():
            row = vox_ref[pl.ds(v0, 1), :]
            o_ref[pl.ds(r0, group), :] = jnp.broadcast_to(
                row, (group, row.shape[1]))

        @pl.when(v0 != v1)
        def _slow():
            def rb(r, c):
                v = idx_ref[0, 0, r]
                o_ref[pl.ds(r, 1), :] = vox_ref[pl.ds(v, 1), :]
                return c
            jax.lax.fori_loop(r0, r0 + group, rb, 0)

        return carry

    jax.lax.fori_loop(0, rows // group, body, 0)


def _mlp1(inp, w1t, b1, w2t, b2):
    n, cin = inp.shape
    gf = w1t.shape[1]
    f1 = w2t.shape[1]
    bn = _pick_block(n, 640)
    grid = n // bn
    return pl.pallas_call(
        _mlp1_kernel,
        grid=(grid,),
        in_specs=[
            pl.BlockSpec((bn, cin), lambda i: (i, 0)),
            pl.BlockSpec((cin, gf), lambda i: (0, 0)),
            pl.BlockSpec((1, gf), lambda i: (0, 0)),
            pl.BlockSpec((gf, f1), lambda i: (0, 0)),
            pl.BlockSpec((1, f1), lambda i: (0, 0)),
        ],
        out_specs=pl.BlockSpec((bn, f1), lambda i: (i, 0)),
        out_shape=jax.ShapeDtypeStruct((n, f1), jnp.float32),
    )(inp, w1t, b1.reshape(1, -1), w2t, b2.reshape(1, -1))


def _voxmm(x, wt, b):
    v, fin = x.shape
    fout = wt.shape[1]
    bv = _pick_block(v, 1000)
    grid = v // bv
    return pl.pallas_call(
        _voxmm_kernel,
        grid=(grid,),
        in_specs=[
            pl.BlockSpec((bv, fin), lambda i: (i, 0)),
            pl.BlockSpec((fin, fout), lambda i: (0, 0)),
            pl.BlockSpec((1, fout), lambda i: (0, 0)),
        ],
        out_specs=pl.BlockSpec((bv, fout), lambda i: (i, 0)),
        out_shape=jax.ShapeDtypeStruct((v, fout), jnp.float32),
    )(x, wt, b.reshape(1, -1))


def _segmax(x, idx3, v, group):
    n, f = x.shape
    nb, _, br = idx3.shape
    return pl.pallas_call(
        functools.partial(_segmax_kernel, rows=br, group=group),
        grid=(nb,),
        in_specs=[
            pl.BlockSpec((1, 1, br), lambda i: (i, 0, 0),
                         memory_space=pltpu.SMEM),
            pl.BlockSpec((br, f), lambda i: (i, 0)),
        ],
        out_specs=pl.BlockSpec((v, f), lambda i: (0, 0)),
        out_shape=jax.ShapeDtypeStruct((v, f), jnp.float32),
    )(idx3, x)


def _gather(vox, idx3, n, group):
    v, f = vox.shape
    nb, _, br = idx3.shape
    return pl.pallas_call(
        functools.partial(_gather_kernel, rows=br, group=group),
        grid=(nb,),
        in_specs=[
            pl.BlockSpec((1, 1, br), lambda i: (i, 0, 0),
                         memory_space=pltpu.SMEM),
            pl.BlockSpec((v, f), lambda i: (0, 0)),
        ],
        out_specs=pl.BlockSpec((br, f), lambda i: (i, 0)),
        out_shape=jax.ShapeDtypeStruct((n, f), jnp.float32),
    )(idx3, vox)


def _sc_gather(vox, idx, n):
    v, f = vox.shape
    cpw = n // _SC_NW
    ch = 400
    while cpw % ch != 0:
        ch -= 8
    nch = cpw // ch
    mesh = plsc.VectorSubcoreMesh(core_axis_name="c", subcore_axis_name="s")

    def body(vox_hbm, idx_hbm, out_hbm, idx_v, rows_v, sem):
        wid = lax.axis_index("s") * _SC_NC + lax.axis_index("c")
        base = wid * cpw
        pltpu.sync_copy(idx_hbm.at[pl.ds(base, cpw)], idx_v)

        def start(c, b):
            return pltpu.async_copy(
                vox_hbm.at[idx_v.at[pl.ds(c * ch, ch)]], rows_v.at[b], sem)

        cps = [start(0, 0), None]
        for c in range(nch):
            b = c % 2
            if c + 1 < nch:
                cps[1 - b] = start(c + 1, 1 - b)
            cps[b].wait()
            pltpu.sync_copy(rows_v.at[b],
                            out_hbm.at[pl.ds(base + c * ch, ch)])

    return pl.kernel(
        body,
        out_type=jax.ShapeDtypeStruct((n, f), jnp.float32),
        mesh=mesh,
        scratch_types=[
            pltpu.VMEM((cpw,), jnp.int32),
            pltpu.VMEM((2, ch, f), jnp.float32),
            pltpu.SemaphoreType.DMA,
        ],
    )(vox, idx)


def _sc_segmax(x, idx, zeros_flat, v):
    n, f = x.shape
    c = n // _SC_NW
    fj = f // 16
    if f == 128:
        r, sw = 400, 128
    else:
        r, sw = 80, 256
    nwin = c // r
    pieces = []
    p = sw
    while p >= 1:
        pieces.append(p)
        p //= 2
    zpieces = []
    p = 8192
    while p >= 1:
        zpieces.append(p)
        p //= 2
    mesh = plsc.VectorSubcoreMesh(core_axis_name="c", subcore_axis_name="s")

    def body(x_hbm, idx_hbm, z_hbm, out_hbm, hacc_hbm, prev_hbm,
             xbf, ibf, win, bnd, sx0, sx1, si0, si1):
        wid = lax.axis_index("s") * _SC_NC + lax.axis_index("c")
        base = wid * c
        zero16 = jnp.zeros((16,), jnp.float32)

        def zero_rows(lo, hi):
            def zrow(i, carry):
                for j in range(fj):
                    win[pl.ds(i * f + 16 * j, 16)] = zero16
                return carry
            lax.fori_loop(lo, hi, zrow, 0)

        zero_rows(0, sw + 1)

        pltpu.sync_copy(
            idx_hbm.at[pl.ds(pl.multiple_of(jnp.maximum(base - 16, 0), 16),
                             16)], bnd)
        prev_last = jnp.where(wid == 0, jnp.int32(-1), bnd[...][15])
        own_lo = prev_last + 1
        pltpu.sync_copy(
            idx_hbm.at[pl.ds(pl.multiple_of(base + c - 16, 16), 16)], bnd)
        last_mine = bnd[...][15]
        own_hi = jnp.where(wid == _SC_NW - 1, jnp.int32(v), last_mine + 1)

        def zfill(lo, hi):
            ln = jnp.clip(jnp.minimum(hi, own_hi) - lo, 0, v)
            off = jnp.int32(0)
            for pc in zpieces:
                pred = (ln & pc) != 0

                @pl.when(pred)
                def _(off=off, pc=pc):
                    pltpu.sync_copy(
                        z_hbm.at[pl.ds(0, pc * f)],
                        out_hbm.at[pl.ds(
                            pl.multiple_of((lo + off) * f, 128), pc * f)])
                off = off + jnp.where(pred, pc, 0)

        def drain_zero(w0):
            ln = jnp.clip(own_hi - w0, 0, sw)
            off = jnp.int32(0)
            for pc in pieces:
                pred = (ln & pc) != 0

                @pl.when(pred)
                def _(off=off, pc=pc):
                    pltpu.sync_copy(
                        win.at[pl.ds(pl.multiple_of(off * f, 128), pc * f)],
                        out_hbm.at[pl.ds(
                            pl.multiple_of((w0 + off) * f, 128), pc * f)])
                off = off + jnp.where(pred, pc, 0)
            zero_rows(0, sw)

        def rmw(rowscalar, accs):
            o = rowscalar * f
            for j in range(fj):
                cur = win[pl.ds(o + 16 * j, 16)]
                win[pl.ds(o + 16 * j, 16)] = jnp.maximum(cur, accs[j])

        def xrow(boff, row):
            return [xbf[boff + row, pl.ds(16 * j, 16)] for j in range(fj)]

        def group_body(b):
            boff = b * r

            def gb(g, w0):
                iv = ibf[pl.ds(b * r + g * 16, 16)]
                g0 = iv[0]
                g15 = iv[15]
                row0 = g * 16

                def cls_u(w0):
                    accs = xrow(boff, row0)
                    for rr in range(1, 16):
                        nxt = xrow(boff, row0 + rr)
                        accs = [jnp.maximum(a, bq)
                                for a, bq in zip(accs, nxt)]
                    trow = jnp.where(g0 == prev_last, sw, g0 - w0)
                    rmw(trow, accs)
                    return w0

                def general(w0):
                    svals = [iv[rr] for rr in range(16)]

                    def rnd(q, carry):
                        w0q, active = carry

                        def do(w):
                            for rr in range(16):
                                s = svals[rr]
                                isp = s == prev_last
                                ok = isp | ((s >= own_lo) & (s >= w)
                                            & (s < w + sw))

                                @pl.when(ok)
                                def _(s=s, rr=rr, w=w, isp=isp):
                                    rmw(jnp.where(isp, sw, s - w),
                                        xrow(boff, row0 + rr))
                            cross = g15 >= w + sw

                            @pl.when(cross)
                            def _(w=w):
                                drain_zero(w)
                            nw0 = g15
                            for rr in range(14, -1, -1):
                                nw0 = jnp.where(svals[rr] >= w + sw,
                                                svals[rr], nw0)

                            @pl.when(cross)
                            def _(w=w, nw0=nw0):
                                zfill(w + sw, nw0)
                            return (jnp.where(cross, nw0, w), cross)

                        def skip(w):
                            return (w, jnp.bool_(False))

                        return lax.cond(active, do, skip, w0q)

                    w, _ = lax.fori_loop(0, 16, rnd,
                                         (w0, jnp.bool_(True)))
                    return w

                is_u = (g0 == g15) & (
                    (g0 == prev_last)
                    | ((g0 >= own_lo) & (g15 < w0 + sw)))
                return lax.cond(is_u, cls_u, general, w0)
            return gb

        def start_w(wi, b, sx, si):
            pltpu.async_copy(
                x_hbm.at[pl.ds(pl.multiple_of(base + wi * r, 8), r), :],
                xbf.at[pl.ds(b * r, r), :], sx)
            pltpu.async_copy(
                idx_hbm.at[pl.ds(pl.multiple_of(base + wi * r, 16), r)],
                ibf.at[pl.ds(b * r, r)], si)

        def wait_w(wi, b, sx, si):
            pltpu.make_async_copy(
                x_hbm.at[pl.ds(pl.multiple_of(base + wi * r, 8), r), :],
                xbf.at[pl.ds(b * r, r), :], sx).wait()
            pltpu.make_async_copy(
                idx_hbm.at[pl.ds(pl.multiple_of(base + wi * r, 16), r)],
                ibf.at[pl.ds(b * r, r)], si).wait()

        start_w(0, 0, sx0, si0)
        start_w(1, 1, sx1, si1)

        def wbody(wi, w0):
            def even(w0):
                wait_w(wi, 0, sx0, si0)
                w0 = lax.fori_loop(0, r // 16, group_body(0), w0)

                @pl.when(wi + 2 < nwin)
                def _():
                    start_w(wi + 2, 0, sx0, si0)
                return w0

            def odd(w0):
                wait_w(wi, 1, sx1, si1)
                w0 = lax.fori_loop(0, r // 16, group_body(1), w0)

                @pl.when(wi + 2 < nwin)
                def _():
                    start_w(wi + 2, 1, sx1, si1)
                return w0

            return lax.cond(wi % 2 == 0, even, odd, w0)

        w0 = lax.fori_loop(0, nwin, wbody, own_lo)

        drain_zero(w0)
        zfill(w0 + sw, own_hi)
        pltpu.sync_copy(
            win.at[pl.ds(pl.multiple_of(sw * f, 128), f)],
            hacc_hbm.at[pl.ds(pl.multiple_of(wid * f, 128), f)])
        bnd[pl.ds(0, 16)] = jnp.full((16,), prev_last, jnp.int32)
        pltpu.sync_copy(bnd,
                        prev_hbm.at[pl.ds(pl.multiple_of(wid * 16, 16), 16)])

    outs = pl.kernel(
        body,
        out_type=(
            jax.ShapeDtypeStruct((v * f,), jnp.float32),
            jax.ShapeDtypeStruct((_SC_NW * f,), jnp.float32),
            jax.ShapeDtypeStruct((_SC_NW * 16,), jnp.int32),
        ),
        mesh=mesh,
        scratch_types=[
            pltpu.VMEM((2 * r, f), jnp.float32),
            pltpu.VMEM((2 * r,), jnp.int32),
            pltpu.VMEM(((sw + 1) * f,), jnp.float32),
            pltpu.VMEM((16,), jnp.int32),
            pltpu.SemaphoreType.DMA,
            pltpu.SemaphoreType.DMA,
            pltpu.SemaphoreType.DMA,
            pltpu.SemaphoreType.DMA,
        ],
    )(x, idx, zeros_flat)
    return outs[0].reshape(v, f), outs[1].reshape(_SC_NW, f), outs[2]


def _fold_kernel(prev_ref, hacc_ref, vox_ref, o_ref, *, nw):
    o_ref[...] = vox_ref[...]

    def fb(k, carry):
        pv = prev_ref[0, 0, k * 16]

        @pl.when(pv >= 0)
        def _():
            o_ref[pl.ds(pv, 1), :] = jnp.maximum(
                o_ref[pl.ds(pv, 1), :], hacc_ref[pl.ds(k, 1), :])
        return carry
    lax.fori_loop(0, nw, fb, 0)


def _fold(prevs, haccs, vox):
    v, f = vox.shape
    nw = haccs.shape[0]
    return pl.pallas_call(
        functools.partial(_fold_kernel, nw=nw),
        grid=(1,),
        in_specs=[
            pl.BlockSpec((1, 1, nw * 16), lambda i: (0, 0, 0),
                         memory_space=pltpu.SMEM),
            pl.BlockSpec((nw, f), lambda i: (0, 0)),
            pl.BlockSpec((v, f), lambda i: (0, 0)),
        ],
        out_specs=pl.BlockSpec((v, f), lambda i: (0, 0)),
        out_shape=jax.ShapeDtypeStruct((v, f), jnp.float32),
    )(prevs.reshape(1, 1, nw * 16), haccs, vox)


def _mlp2(pg, pf, w3at, w3bt, b3, w4t, b4):
    n, f1 = pf.shape
    oc = w4t.shape[1]
    bn = _pick_block(n, 640)
    grid = n // bn
    return pl.pallas_call(
        _mlp2_kernel,
        grid=(grid,),
        in_specs=[
            pl.BlockSpec((bn, f1), lambda i: (i, 0)),
            pl.BlockSpec((bn, f1), lambda i: (i, 0)),
            pl.BlockSpec((f1, oc), lambda i: (0, 0)),
            pl.BlockSpec((f1, oc), lambda i: (0, 0)),
            pl.BlockSpec((1, oc), lambda i: (0, 0)),
            pl.BlockSpec((oc, oc), lambda i: (0, 0)),
            pl.BlockSpec((1, oc), lambda i: (0, 0)),
        ],
        out_specs=pl.BlockSpec((bn, oc), lambda i: (i, 0)),
        out_shape=jax.ShapeDtypeStruct((n, oc), jnp.float32),
    )(pg, pf, w3at, w3bt, b3.reshape(1, -1), w4t, b4.reshape(1, -1))


def _forward(inp_feat, vox2point_idx, W1, b1, W2, b2, Wv1, bv1,
             W3, b3, W4, b4, Wv2, bv2, v):
    n = inp_feat.shape[0]
    f1 = W2.shape[0]
    idx = vox2point_idx.astype(jnp.int32)

    br = _pick_block(n, 3200)
    idx3 = idx.reshape(n // br, 1, br)

    cin = inp_feat.shape[1]
    cpad = (-cin) % 8
    inp_p = jnp.pad(inp_feat, ((0, 0), (0, cpad)))
    w1t = jnp.pad(W1.T, ((0, cpad), (0, 0)))

    pf2 = _mlp1(inp_p, w1t, b1, W2.T, b2)
    z128 = jnp.zeros((v * f1,), jnp.float32)
    vm1, ha1, pv1 = _sc_segmax(pf2, idx, z128, v)
    vox1 = _fold(pv1, ha1, vm1)
    vox1r = _voxmm(vox1, Wv1.T, bv1)
    pgf = _sc_gather(vox1r, idx, n)
    pf5 = _mlp2(pgf, pf2, W3[:, :f1].T, W3[:, f1:].T, b3, W4.T, b4)
    oc = W3.shape[0]
    z256 = jnp.zeros((v * oc,), jnp.float32)
    vm2, ha2, pv2 = _sc_segmax(pf5, idx, z256, v)
    vox2 = _fold(pv2, ha2, vm2)
    return _voxmm(vox2, Wv2.T, bv2)


def kernel(inp_feat, vox2point_idx, W1, b1, W2, b2, Wv1, bv1,
           W3, b3, W4, b4, Wv2, bv2):
    return _forward(inp_feat, vox2point_idx, W1, b1, W2, b2, Wv1, bv1,
                    W3, b3, W4, b4, Wv2, bv2, v=10000)
